# Initial kernel scaffold; baseline (speedup 1.0000x reference)
import jax
import jax.numpy as jnp
from jax.experimental import pallas as pl


def kernel(x_region, x_subject, edge_index_rr, edge_index_rs, W_gcn1, b_gcn1, W_sage_l1, W_sage_r1, b_sage1, W_gcn2, b_gcn2, W_sage_l2, W_sage_r2, b_sage2, W_lin, b_lin):
    raise NotImplementedError("write your pallas kernel here")


# speedup vs baseline: 8.0266x; 8.0266x over previous
import functools

import jax
import jax.numpy as jnp
from jax import lax
from jax.experimental import pallas as pl
from jax.experimental.pallas import tpu as pltpu
from jax.experimental.pallas import tpu_sc as plsc

NR = 10000
NR_P = 10240
NSUB = 1000
NSUB_P = 1024
D = 128
H = 128
OUTD = 32
ERR = 320000
ERS = 160000

NC = 2
NS = 16
NW = NC * NS

CRR = 80
CRS = 40
RR_PER = ERR // NW
RS_PER = ERS // NW
NCH_RR = RR_PER // CRR
NCH_RS = RS_PER // CRS
ROWS_R = NR_P // NS
ROWS_S = NSUB_P // NS

_MESH = plsc.VectorSubcoreMesh(
    core_axis_name="c", subcore_axis_name="s", num_cores=NC, num_subcores=NS
)

f32 = jnp.float32


def _sc_pass1_body(dst_rr, src_rs, dst_rs, x_region, z_deg, z_cnt, z_sum,
                   ones_hbm, deg_out, cnt_out, sum1_out,
                   deg_acc, cnt_acc, sum_acc,
                   idx80, idx40d, idx40s, ones_v, rows40, sem):
    c = lax.axis_index("c")
    s = lax.axis_index("s")
    wid = c * NS + s
    pltpu.sync_copy(ones_hbm, ones_v)
    pltpu.sync_copy(z_deg, deg_acc.at[pl.ds(s * ROWS_R, ROWS_R)])
    pltpu.sync_copy(z_cnt, cnt_acc.at[pl.ds(s * ROWS_S, ROWS_S)])
    pltpu.sync_copy(z_sum, sum_acc.at[pl.ds(s * ROWS_S, ROWS_S)])
    plsc.subcore_barrier()

    base_rr = wid * RR_PER

    def rr_body(j, carry):
        off = base_rr + j * CRR
        pltpu.sync_copy(dst_rr.at[pl.ds(off, CRR)], idx80)
        pltpu.sync_copy(ones_v, deg_acc.at[idx80], add=True)
        return carry

    lax.fori_loop(0, NCH_RR, rr_body, 0)

    base_rs = wid * RS_PER

    def rs_body(j, carry):
        off = base_rs + j * CRS
        pltpu.sync_copy(dst_rs.at[pl.ds(off, CRS)], idx40d)
        pltpu.sync_copy(src_rs.at[pl.ds(off, CRS)], idx40s)
        pltpu.async_copy(x_region.at[idx40s], rows40, sem).wait()
        pltpu.sync_copy(ones_v.at[pl.ds(0, CRS)], cnt_acc.at[idx40d], add=True)
        pltpu.sync_copy(rows40, sum_acc.at[idx40d], add=True)
        return carry

    lax.fori_loop(0, NCH_RS, rs_body, 0)
    plsc.subcore_barrier()

    pltpu.sync_copy(deg_acc.at[pl.ds(s * ROWS_R, ROWS_R)],
                    deg_out.at[c].at[pl.ds(s * ROWS_R, ROWS_R)])
    pltpu.sync_copy(cnt_acc.at[pl.ds(s * ROWS_S, ROWS_S)],
                    cnt_out.at[c].at[pl.ds(s * ROWS_S, ROWS_S)])
    pltpu.sync_copy(sum_acc.at[pl.ds(s * ROWS_S, ROWS_S)],
                    sum1_out.at[c].at[pl.ds(s * ROWS_S, ROWS_S)])


_sc_pass1 = pl.kernel(
    _sc_pass1_body,
    out_type=(
        jax.ShapeDtypeStruct((NC, NR_P, 16), f32),
        jax.ShapeDtypeStruct((NC, NSUB_P, 16), f32),
        jax.ShapeDtypeStruct((NC, NSUB_P, D), f32),
    ),
    mesh=_MESH,
    scratch_types=(
        pltpu.VMEM_SHARED((NR_P, 16), f32),
        pltpu.VMEM_SHARED((NSUB_P, 16), f32),
        pltpu.VMEM_SHARED((NSUB_P, D), f32),
        pltpu.VMEM((CRR,), jnp.int32),
        pltpu.VMEM((CRS,), jnp.int32),
        pltpu.VMEM((CRS,), jnp.int32),
        pltpu.VMEM((CRR, 16), f32),
        pltpu.VMEM((CRS, D), f32),
        pltpu.SemaphoreType.DMA,
    ),
)


def _sc_agg_body(src_rr, dst_rr, y, z_agg, agg_out,
                 agg_acc, idx_s, idx_d, rows, sem):
    c = lax.axis_index("c")
    s = lax.axis_index("s")
    wid = c * NS + s
    pltpu.sync_copy(z_agg, agg_acc.at[pl.ds(s * ROWS_R, ROWS_R)])
    plsc.subcore_barrier()

    base = wid * RR_PER

    def rr_body(j, carry):
        off = base + j * CRR
        pltpu.sync_copy(src_rr.at[pl.ds(off, CRR)], idx_s)
        pltpu.async_copy(y.at[idx_s], rows, sem).wait()
        pltpu.sync_copy(dst_rr.at[pl.ds(off, CRR)], idx_d)
        pltpu.sync_copy(rows, agg_acc.at[idx_d], add=True)
        return carry

    lax.fori_loop(0, NCH_RR, rr_body, 0)
    plsc.subcore_barrier()
    pltpu.sync_copy(agg_acc.at[pl.ds(s * ROWS_R, ROWS_R)],
                    agg_out.at[c].at[pl.ds(s * ROWS_R, ROWS_R)])


_sc_agg = pl.kernel(
    _sc_agg_body,
    out_type=jax.ShapeDtypeStruct((NC, NR_P, D), f32),
    mesh=_MESH,
    scratch_types=(
        pltpu.VMEM_SHARED((NR_P, D), f32),
        pltpu.VMEM((CRR,), jnp.int32),
        pltpu.VMEM((CRR,), jnp.int32),
        pltpu.VMEM((CRR, D), f32),
        pltpu.SemaphoreType.DMA,
    ),
)


def _sc_agg2_body(src_rr, dst_rr, y2, src_rs, dst_rs, r1, z_agg, z_sum,
                  agg_out, sum2_out,
                  agg_acc, sum_acc, idx_s, idx_d, idx40s, idx40d,
                  rows, rows40, sem):
    c = lax.axis_index("c")
    s = lax.axis_index("s")
    wid = c * NS + s
    pltpu.sync_copy(z_agg, agg_acc.at[pl.ds(s * ROWS_R, ROWS_R)])
    pltpu.sync_copy(z_sum, sum_acc.at[pl.ds(s * ROWS_S, ROWS_S)])
    plsc.subcore_barrier()

    base = wid * RR_PER

    def rr_body(j, carry):
        off = base + j * CRR
        pltpu.sync_copy(src_rr.at[pl.ds(off, CRR)], idx_s)
        pltpu.async_copy(y2.at[idx_s], rows, sem).wait()
        pltpu.sync_copy(dst_rr.at[pl.ds(off, CRR)], idx_d)
        pltpu.sync_copy(rows, agg_acc.at[idx_d], add=True)
        return carry

    lax.fori_loop(0, NCH_RR, rr_body, 0)

    base_rs = wid * RS_PER

    def rs_body(j, carry):
        off = base_rs + j * CRS
        pltpu.sync_copy(src_rs.at[pl.ds(off, CRS)], idx40s)
        pltpu.async_copy(r1.at[idx40s], rows40, sem).wait()
        pltpu.sync_copy(dst_rs.at[pl.ds(off, CRS)], idx40d)
        pltpu.sync_copy(rows40, sum_acc.at[idx40d], add=True)
        return carry

    lax.fori_loop(0, NCH_RS, rs_body, 0)
    plsc.subcore_barrier()
    pltpu.sync_copy(agg_acc.at[pl.ds(s * ROWS_R, ROWS_R)],
                    agg_out.at[c].at[pl.ds(s * ROWS_R, ROWS_R)])
    pltpu.sync_copy(sum_acc.at[pl.ds(s * ROWS_S, ROWS_S)],
                    sum2_out.at[c].at[pl.ds(s * ROWS_S, ROWS_S)])


_sc_agg2 = pl.kernel(
    _sc_agg2_body,
    out_type=(
        jax.ShapeDtypeStruct((NC, NR_P, D), f32),
        jax.ShapeDtypeStruct((NC, NSUB_P, D), f32),
    ),
    mesh=_MESH,
    scratch_types=(
        pltpu.VMEM_SHARED((NR_P, D), f32),
        pltpu.VMEM_SHARED((NSUB_P, D), f32),
        pltpu.VMEM((CRR,), jnp.int32),
        pltpu.VMEM((CRR,), jnp.int32),
        pltpu.VMEM((CRS,), jnp.int32),
        pltpu.VMEM((CRS,), jnp.int32),
        pltpu.VMEM((CRR, D), f32),
        pltpu.VMEM((CRS, D), f32),
        pltpu.SemaphoreType.DMA,
    ),
)


def _tc_a_body(deg_ref, cnt_ref, sum1_ref, xr_ref, xs_ref, wl_ref, wr_ref,
               b_ref, y1_ref, dis_ref, s1_ref, invc_ref):
    deg = deg_ref[0, :NR, 0:1] + deg_ref[1, :NR, 0:1] + 1.0
    dis = lax.rsqrt(deg)
    dis_ref[...] = dis
    y1_ref[...] = xr_ref[...] * dis
    cnt = cnt_ref[0, :NSUB, 0:1] + cnt_ref[1, :NSUB, 0:1]
    invc = 1.0 / jnp.maximum(cnt, 1.0)
    invc_ref[...] = invc
    mean1 = (sum1_ref[0, :NSUB, :] + sum1_ref[1, :NSUB, :]) * invc
    s1_ref[...] = (
        jnp.dot(mean1, wl_ref[...], preferred_element_type=f32)
        + jnp.dot(xs_ref[...], wr_ref[...], preferred_element_type=f32)
        + b_ref[...]
    )


def _tc_b_body(agg_ref, y1_ref, dis_ref, w_ref, b_ref, r1_ref, y2_ref):
    g = (agg_ref[0, :NR, :] + agg_ref[1, :NR, :] + y1_ref[...]) * dis_ref[...]
    r1 = jnp.dot(g, w_ref[...], preferred_element_type=f32) + b_ref[...]
    r1_ref[...] = r1
    y2_ref[...] = r1 * dis_ref[...]


def _tc_c_body(agg_ref, y2_ref, dis_ref, w2_ref, b2_ref,
               sum2_ref, invc_ref, s1_ref, wl2_ref, wr2_ref, bs2_ref,
               wlin_ref, blin_ref, outr_ref, outs_ref):
    g = (agg_ref[0, :NR, :] + agg_ref[1, :NR, :] + y2_ref[...]) * dis_ref[...]
    r2 = jnp.dot(g, w2_ref[...], preferred_element_type=f32) + b2_ref[...]
    outr_ref[...] = (
        jnp.dot(r2, wlin_ref[...], preferred_element_type=f32) + blin_ref[...]
    )
    mean2 = (sum2_ref[0, :NSUB, :] + sum2_ref[1, :NSUB, :]) * invc_ref[...]
    s2 = (
        jnp.dot(mean2, wl2_ref[...], preferred_element_type=f32)
        + jnp.dot(s1_ref[...], wr2_ref[...], preferred_element_type=f32)
        + bs2_ref[...]
    )
    outs_ref[...] = (
        jnp.dot(s2, wlin_ref[...], preferred_element_type=f32) + blin_ref[...]
    )


def kernel(x_region, x_subject, edge_index_rr, edge_index_rs,
           W_gcn1, b_gcn1, W_sage_l1, W_sage_r1, b_sage1,
           W_gcn2, b_gcn2, W_sage_l2, W_sage_r2, b_sage2,
           W_lin, b_lin):
    src_rr = edge_index_rr[0]
    dst_rr = edge_index_rr[1]
    src_rs = edge_index_rs[0]
    dst_rs = edge_index_rs[1]

    z_deg = jnp.zeros((ROWS_R, 16), f32)
    z_cnt = jnp.zeros((ROWS_S, 16), f32)
    z_sum = jnp.zeros((ROWS_S, D), f32)
    z_agg = jnp.zeros((ROWS_R, D), f32)
    ones80 = jnp.ones((CRR, 16), f32)

    deg_p, cnt_p, sum1_p = _sc_pass1(
        dst_rr, src_rs, dst_rs, x_region, z_deg, z_cnt, z_sum, ones80
    )

    y1, dis, s1, invc = pl.pallas_call(
        _tc_a_body,
        out_shape=(
            jax.ShapeDtypeStruct((NR, D), f32),
            jax.ShapeDtypeStruct((NR, 1), f32),
            jax.ShapeDtypeStruct((NSUB, H), f32),
            jax.ShapeDtypeStruct((NSUB, 1), f32),
        ),
    )(deg_p, cnt_p, sum1_p, x_region, x_subject,
      W_sage_l1, W_sage_r1, b_sage1.reshape(1, H))

    agg1_p = _sc_agg(src_rr, dst_rr, y1, z_agg)

    r1, y2 = pl.pallas_call(
        _tc_b_body,
        out_shape=(
            jax.ShapeDtypeStruct((NR, H), f32),
            jax.ShapeDtypeStruct((NR, H), f32),
        ),
    )(agg1_p, y1, dis, W_gcn1, b_gcn1.reshape(1, H))

    agg2_p, sum2_p = _sc_agg2(src_rr, dst_rr, y2, src_rs, dst_rs, r1,
                              z_agg, z_sum)

    out_region, out_subject = pl.pallas_call(
        _tc_c_body,
        out_shape=(
            jax.ShapeDtypeStruct((NR, OUTD), f32),
            jax.ShapeDtypeStruct((NSUB, OUTD), f32),
        ),
    )(agg2_p, y2, dis, W_gcn2, b_gcn2.reshape(1, H),
      sum2_p, invc, s1, W_sage_l2, W_sage_r2, b_sage2.reshape(1, H),
      W_lin, b_lin.reshape(1, OUTD))

    return (out_region, out_subject)
